# Initial kernel scaffold; baseline (speedup 1.0000x reference)
import functools

import jax
import jax.numpy as jnp
from jax import lax
from jax.experimental import pallas as pl
from jax.experimental.pallas import tpu as pltpu
from jax.experimental.pallas import tpu_sc as plsc

N = 50000
E = 800000
H = 64

N_PAD = 50176
E_PAD = 802816
SENTINEL = N

NUM_SC = 2
NUM_SUBCORES = 16
NW = NUM_SC * NUM_SUBCORES
CH = 128

BE = 2048
BN = 1024

_mesh = plsc.VectorSubcoreMesh(core_axis_name="c", subcore_axis_name="s")


def _dot(a, b):
    return jnp.dot(a, b, preferred_element_type=jnp.float32)


def _mlp3(feats, p3, block_rows):
    (W1, b1), (W2, b2), (W3, b3) = p3
    R, Fin = feats.shape
    F1, F2, F3 = W1.shape[1], W2.shape[1], W3.shape[1]

    def body(f, w1, b1r, w2, b2r, w3, b3r, o):
        h = jnp.maximum(_dot(f[...], w1[...]) + b1r[...], 0.0)
        h = jnp.maximum(_dot(h, w2[...]) + b2r[...], 0.0)
        o[...] = _dot(h, w3[...]) + b3r[...]

    return pl.pallas_call(
        body,
        grid=(R // block_rows,),
        in_specs=[
            pl.BlockSpec((block_rows, Fin), lambda i: (i, 0)),
            pl.BlockSpec((Fin, F1), lambda i: (0, 0)),
            pl.BlockSpec((1, F1), lambda i: (0, 0)),
            pl.BlockSpec((F1, F2), lambda i: (0, 0)),
            pl.BlockSpec((1, F2), lambda i: (0, 0)),
            pl.BlockSpec((F2, F3), lambda i: (0, 0)),
            pl.BlockSpec((1, F3), lambda i: (0, 0)),
        ],
        out_specs=pl.BlockSpec((block_rows, F3), lambda i: (i, 0)),
        out_shape=jax.ShapeDtypeStruct((R, F3), jnp.float32),
    )(feats, W1, b1.reshape(1, -1), W2, b2.reshape(1, -1), W3,
      b3.reshape(1, -1))


def _edge_block(e, xs, xd, p3):
    (W1, b1), (W2, b2), (W3, b3) = p3
    W1e, W1s, W1d = W1[:H], W1[H:2 * H], W1[2 * H:]

    def body(er, xsr, xdr, w1e, w1s, w1d, b1r, w2, b2r, w3, b3r, o):
        h = jnp.maximum(
            _dot(er[...], w1e[...]) + _dot(xsr[...], w1s[...])
            + _dot(xdr[...], w1d[...]) + b1r[...], 0.0)
        h = jnp.maximum(_dot(h, w2[...]) + b2r[...], 0.0)
        o[...] = er[...] + _dot(h, w3[...]) + b3r[...]

    rows = pl.BlockSpec((BE, H), lambda i: (i, 0))
    wspec = pl.BlockSpec((H, H), lambda i: (0, 0))
    bspec = pl.BlockSpec((1, H), lambda i: (0, 0))
    return pl.pallas_call(
        body,
        grid=(E_PAD // BE,),
        in_specs=[rows, rows, rows, wspec, wspec, wspec, bspec, wspec, bspec,
                  wspec, bspec],
        out_specs=rows,
        out_shape=jax.ShapeDtypeStruct((E_PAD, H), jnp.float32),
    )(e, xs, xd, W1e, W1s, W1d, b1.reshape(1, -1), W2, b2.reshape(1, -1),
      W3, b3.reshape(1, -1))


def _node_block(x, agg, p3):
    (W1, b1), (W2, b2), (W3, b3) = p3
    W1x, W1a = W1[:H], W1[H:]

    def body(xr, ar, w1x, w1a, b1r, w2, b2r, w3, b3r, o):
        h = jnp.maximum(
            _dot(xr[...], w1x[...]) + _dot(ar[...], w1a[...]) + b1r[...], 0.0)
        h = jnp.maximum(_dot(h, w2[...]) + b2r[...], 0.0)
        o[...] = xr[...] + _dot(h, w3[...]) + b3r[...]

    rows = pl.BlockSpec((BN, H), lambda i: (i, 0))
    wspec = pl.BlockSpec((H, H), lambda i: (0, 0))
    bspec = pl.BlockSpec((1, H), lambda i: (0, 0))
    return pl.pallas_call(
        body,
        grid=(N_PAD // BN,),
        in_specs=[rows, rows, wspec, wspec, bspec, wspec, bspec, wspec, bspec],
        out_specs=rows,
        out_shape=jax.ShapeDtypeStruct((N_PAD, H), jnp.float32),
    )(x, agg, W1x, W1a, b1.reshape(1, -1), W2, b2.reshape(1, -1), W3,
      b3.reshape(1, -1))


_EW = E_PAD // NW
_NCH_G = _EW // CH
_ES = E_PAD // NUM_SUBCORES
_NCH_S = _ES // CH
_ZR = N_PAD // NUM_SUBCORES
HC = H // NUM_SC


@functools.partial(
    pl.kernel,
    mesh=_mesh,
    out_type=[jax.ShapeDtypeStruct((E_PAD, H), jnp.float32),
              jax.ShapeDtypeStruct((E_PAD, H), jnp.float32)],
    scratch_types=[
        pltpu.VMEM((CH,), jnp.int32),
        pltpu.VMEM((CH,), jnp.int32),
        pltpu.VMEM((CH, H), jnp.float32),
        pltpu.VMEM((CH, H), jnp.float32),
        pltpu.SemaphoreType.DMA,
        pltpu.SemaphoreType.DMA,
    ],
)
def _sc_gather(x_hbm, src_hbm, dst_hbm, xs_hbm, xd_hbm,
               si_v, di_v, sr_v, dr_v, sem_s, sem_d):
    wid = lax.axis_index("s") * NUM_SC + lax.axis_index("c")
    base = wid * _EW

    @pl.loop(0, _NCH_G)
    def _(j):
        off = base + j * CH
        pltpu.sync_copy(src_hbm.at[pl.ds(off, CH)], si_v)
        pltpu.sync_copy(dst_hbm.at[pl.ds(off, CH)], di_v)
        cs = pltpu.async_copy(x_hbm.at[si_v], sr_v, sem_s)
        cd = pltpu.async_copy(x_hbm.at[di_v], dr_v, sem_d)
        cs.wait()
        cd.wait()
        pltpu.sync_copy(sr_v, xs_hbm.at[pl.ds(off, CH)])
        pltpu.sync_copy(dr_v, xd_hbm.at[pl.ds(off, CH)])


@functools.partial(
    pl.kernel,
    mesh=_mesh,
    out_type=jax.ShapeDtypeStruct((N_PAD, H), jnp.float32),
    scratch_types=[
        pltpu.VMEM((CH,), jnp.int32),
        pltpu.VMEM((CH, HC), jnp.float32),
        pltpu.VMEM_SHARED((N_PAD, HC), jnp.float32),
    ],
)
def _sc_segment_sum(e_hbm, dst_hbm, zeros_hbm, agg_hbm, idx_v, rows_v, acc_sh):
    cid = lax.axis_index("c")
    sid = lax.axis_index("s")
    col = cid * HC
    pltpu.sync_copy(zeros_hbm.at[pl.ds(sid * _ZR, _ZR)],
                    acc_sh.at[pl.ds(sid * _ZR, _ZR)])
    plsc.subcore_barrier()

    base = sid * _ES

    @pl.loop(0, _NCH_S)
    def _(j):
        off = base + j * CH
        pltpu.sync_copy(dst_hbm.at[pl.ds(off, CH)], idx_v)
        pltpu.sync_copy(e_hbm.at[pl.ds(off, CH), pl.ds(col, HC)], rows_v)
        pltpu.sync_copy(rows_v, acc_sh.at[idx_v], add=True)

    plsc.subcore_barrier()
    pltpu.sync_copy(acc_sh.at[pl.ds(sid * _ZR, _ZR)],
                    agg_hbm.at[pl.ds(sid * _ZR, _ZR), pl.ds(col, HC)])


def kernel(node_features, edge_features, edge_index, params):
    nf = jnp.pad(node_features, ((0, N_PAD - N), (0, 0)))
    ef = jnp.pad(edge_features, ((0, E_PAD - E), (0, 0)))
    src = jnp.pad(edge_index[0], (0, E_PAD - E))
    dst = jnp.pad(edge_index[1], (0, E_PAD - E), constant_values=SENTINEL)
    zeros = jnp.zeros((N_PAD, HC), jnp.float32)

    x = _mlp3(nf, params["node_enc"], BN)
    e = _mlp3(ef, params["edge_enc"], BE)
    for blk in params["blocks"]:
        xs, xd = _sc_gather(x, src, dst)
        e = _edge_block(e, xs, xd, blk["edge"])
        agg = _sc_segment_sum(e, dst, zeros)
        x = _node_block(x, agg, blk["node"])
    out = _mlp3(x, params["decoder"], BN)
    return out[:N]


# speedup vs baseline: 1.0418x; 1.0418x over previous
import functools

import jax
import jax.numpy as jnp
from jax import lax
from jax.experimental import pallas as pl
from jax.experimental.pallas import tpu as pltpu
from jax.experimental.pallas import tpu_sc as plsc

N = 50000
E = 800000
H = 64

N_PAD = 50176
E_PAD = 802816
SENTINEL = N

NUM_SC = 2
NUM_SUBCORES = 16
NW = NUM_SC * NUM_SUBCORES
CH = 128
NB_PAD = 53248

BE = 2048
SUB = 128
BN = 1024

_mesh = plsc.VectorSubcoreMesh(core_axis_name="c", subcore_axis_name="s")


def _dot(a, b):
    return jnp.dot(a, b, preferred_element_type=jnp.float32)


def _mlp3(feats, p3, block_rows):
    (W1, b1), (W2, b2), (W3, b3) = p3
    R, Fin = feats.shape
    F1, F2, F3 = W1.shape[1], W2.shape[1], W3.shape[1]

    def body(f, w1, b1r, w2, b2r, w3, b3r, o):
        h = jnp.maximum(_dot(f[...], w1[...]) + b1r[...], 0.0)
        h = jnp.maximum(_dot(h, w2[...]) + b2r[...], 0.0)
        o[...] = _dot(h, w3[...]) + b3r[...]

    return pl.pallas_call(
        body,
        grid=(R // block_rows,),
        in_specs=[
            pl.BlockSpec((block_rows, Fin), lambda i: (i, 0)),
            pl.BlockSpec((Fin, F1), lambda i: (0, 0)),
            pl.BlockSpec((1, F1), lambda i: (0, 0)),
            pl.BlockSpec((F1, F2), lambda i: (0, 0)),
            pl.BlockSpec((1, F2), lambda i: (0, 0)),
            pl.BlockSpec((F2, F3), lambda i: (0, 0)),
            pl.BlockSpec((1, F3), lambda i: (0, 0)),
        ],
        out_specs=pl.BlockSpec((block_rows, F3), lambda i: (i, 0)),
        out_shape=jax.ShapeDtypeStruct((R, F3), jnp.float32),
    )(feats, W1, b1.reshape(1, -1), W2, b2.reshape(1, -1), W3,
      b3.reshape(1, -1))


def _proj_pair(x, Ws, Wd):
    def body(xr, ws, wd, o):
        o[:, :H] = _dot(xr[...], ws[...])
        o[:, H:] = _dot(xr[...], wd[...])

    rows = pl.BlockSpec((BN, H), lambda i: (i, 0))
    wspec = pl.BlockSpec((H, H), lambda i: (0, 0))
    return pl.pallas_call(
        body,
        grid=(N_PAD // BN,),
        in_specs=[rows, wspec, wspec],
        out_specs=pl.BlockSpec((BN, 2 * H), lambda i: (i, 0)),
        out_shape=jax.ShapeDtypeStruct((N_PAD, 2 * H), jnp.float32),
    )(x, Ws, Wd)


def _edge_block(e, S, D, p3):
    (W1, b1), (W2, b2), (W3, b3) = p3
    W1e = W1[:H]

    def body(er, sr, dr, w1e, b1r, w2, b2r, w3, b3r, oe, ot, carry):
        @pl.when(pl.program_id(0) == 0)
        def _():
            carry[...] = jnp.zeros_like(carry)

        h = jnp.maximum(
            _dot(er[...], w1e[...]) + sr[:, :H] + dr[:, H:] + b1r[...], 0.0)
        h = jnp.maximum(_dot(h, w2[...]) + b2r[...], 0.0)
        e_new = er[...] + _dot(h, w3[...]) + b3r[...]
        oe[...] = e_new

        s = e_new
        k = 1
        while k < BE:
            s = s + jnp.concatenate(
                [jnp.zeros((k, H), jnp.float32), s[:-k]], axis=0)
            k *= 2
        ot[:, :H] = s - e_new
        ot[:, H:] = jnp.broadcast_to(carry[...], (BE, H))
        carry[...] = carry[...] + s[BE - 1:BE, :]

    rows = pl.BlockSpec((BE, H), lambda i: (i, 0))
    prows = pl.BlockSpec((BE, 2 * H), lambda i: (i, 0))
    wspec = pl.BlockSpec((H, H), lambda i: (0, 0))
    bspec = pl.BlockSpec((1, H), lambda i: (0, 0))
    return pl.pallas_call(
        body,
        grid=(E_PAD // BE,),
        in_specs=[rows, prows, prows, wspec, bspec, wspec, bspec,
                  wspec, bspec],
        out_specs=[rows, prows],
        out_shape=[jax.ShapeDtypeStruct((E_PAD, H), jnp.float32),
                   jax.ShapeDtypeStruct((E_PAD, 2 * H), jnp.float32)],
        scratch_shapes=[pltpu.VMEM((1, H), jnp.float32)],
    )(e, S, D, W1e, b1.reshape(1, -1), W2, b2.reshape(1, -1),
      W3, b3.reshape(1, -1))


def _node_block(x, A0, A1, p3):
    (W1, b1), (W2, b2), (W3, b3) = p3
    W1x, W1a = W1[:H], W1[H:]

    def body(xr, a0r, a1r, w1x, w1a, b1r, w2, b2r, w3, b3r, o):
        agg = (a1r[:, :H] - a0r[:, :H]) + (a1r[:, H:] - a0r[:, H:])
        h = jnp.maximum(
            _dot(xr[...], w1x[...]) + _dot(agg, w1a[...]) + b1r[...], 0.0)
        h = jnp.maximum(_dot(h, w2[...]) + b2r[...], 0.0)
        o[...] = xr[...] + _dot(h, w3[...]) + b3r[...]

    rows = pl.BlockSpec((BN, H), lambda i: (i, 0))
    arows = pl.BlockSpec((BN, 2 * H), lambda i: (i, 0))
    wspec = pl.BlockSpec((H, H), lambda i: (0, 0))
    bspec = pl.BlockSpec((1, H), lambda i: (0, 0))
    return pl.pallas_call(
        body,
        grid=(N_PAD // BN,),
        in_specs=[rows, arows, arows, wspec, wspec, bspec, wspec, bspec,
                  wspec, bspec],
        out_specs=rows,
        out_shape=jax.ShapeDtypeStruct((N_PAD, H), jnp.float32),
    )(x, A0, A1, W1x, W1a, b1.reshape(1, -1), W2, b2.reshape(1, -1), W3,
      b3.reshape(1, -1))


@functools.lru_cache(maxsize=None)
def _make_gather(n_idx, table_rows):
    per_tile = n_idx // NW
    n_chunks = per_tile // CH

    @functools.partial(
        pl.kernel,
        mesh=_mesh,
        out_type=[jax.ShapeDtypeStruct((n_idx, 2 * H), jnp.float32),
                  jax.ShapeDtypeStruct((n_idx, 2 * H), jnp.float32)],
        scratch_types=[
            pltpu.VMEM((CH,), jnp.int32),
            pltpu.VMEM((CH,), jnp.int32),
            pltpu.VMEM((CH, 2 * H), jnp.float32),
            pltpu.VMEM((CH, 2 * H), jnp.float32),
            pltpu.SemaphoreType.DMA,
            pltpu.SemaphoreType.DMA,
        ],
    )
    def gather2(pk_hbm, ia_hbm, ib_hbm, a_hbm, b_hbm,
                ai_v, bi_v, ar_v, br_v, sem_a, sem_b):
        wid = lax.axis_index("s") * NUM_SC + lax.axis_index("c")
        base = wid * per_tile

        @pl.loop(0, n_chunks)
        def _(j):
            off = base + j * CH
            pltpu.sync_copy(ia_hbm.at[pl.ds(off, CH)], ai_v)
            pltpu.sync_copy(ib_hbm.at[pl.ds(off, CH)], bi_v)
            ca = pltpu.async_copy(pk_hbm.at[ai_v], ar_v, sem_a)
            cb = pltpu.async_copy(pk_hbm.at[bi_v], br_v, sem_b)
            ca.wait()
            cb.wait()
            pltpu.sync_copy(ar_v, a_hbm.at[pl.ds(off, CH)])
            pltpu.sync_copy(br_v, b_hbm.at[pl.ds(off, CH)])

    return gather2


def kernel(node_features, edge_features, edge_index, params):
    nf = jnp.pad(node_features, ((0, N_PAD - N), (0, 0)))
    ef = jnp.pad(edge_features, ((0, E_PAD - E), (0, 0)))
    src = jnp.pad(edge_index[0], (0, E_PAD - E))
    dst = jnp.pad(edge_index[1], (0, E_PAD - E), constant_values=SENTINEL)

    order = jnp.argsort(dst)
    src = src[order]
    dst = dst[order]
    ef = ef[order]

    rs = jnp.searchsorted(dst, jnp.arange(N_PAD + 1, dtype=jnp.int32))
    rs = jnp.minimum(rs, E_PAD - 1).astype(jnp.int32)
    rs0 = jnp.pad(rs[:N_PAD], (0, NB_PAD - N_PAD))
    rs1 = jnp.pad(rs[1:N_PAD + 1], (0, NB_PAD - N_PAD))

    gather_e = _make_gather(E_PAD, N_PAD)
    gather_n = _make_gather(NB_PAD, E_PAD)

    x = _mlp3(nf, params["node_enc"], BN)
    e = _mlp3(ef, params["edge_enc"], BE)
    for blk in params["blocks"]:
        W1 = blk["edge"][0][0]
        pk = _proj_pair(x, W1[H:2 * H], W1[2 * H:])
        S, D = gather_e(pk, src, dst)
        e, tab = _edge_block(e, S, D, blk["edge"])
        A0, A1 = gather_n(tab, rs0, rs1)
        x = _node_block(x, A0, A1, blk["node"])
    out = _mlp3(x, params["decoder"], BN)
    return out[:N]
